# baseline (speedup 1.0000x reference)
import jax
import jax.numpy as jnp
from jax.experimental import pallas as pl
from jax.experimental.pallas import tpu as pltpu

VOCAB = 100000
D = 64
NCTX = 20
HID = 128
VB = 8192
NB = (VOCAB + VB - 1) // VB


def _logits_kernel(idx_ref, emb_hbm, w1_ref, b1_ref, w2_ref, b2_ref,
                   out_ref, stats_ref, rows_ref, row_sem):
    i = pl.program_id(0)

    copies = [
        pltpu.make_async_copy(
            emb_hbm.at[pl.ds(idx_ref[k], 1), :],
            rows_ref.at[pl.ds(k, 1), :],
            row_sem,
        )
        for k in range(NCTX)
    ]
    for c in copies:
        c.start()
    for c in copies:
        c.wait()
    h = b1_ref[...]
    for k in range(NCTX):
        h = h + jnp.dot(rows_ref[pl.ds(k, 1), :],
                        w1_ref[pl.ds(k * D, D), :],
                        preferred_element_type=jnp.float32)
    h = jnp.maximum(h, 0.0)

    z = jnp.dot(h, w2_ref[...],
                preferred_element_type=jnp.float32) + b2_ref[...]
    col = i * VB + jax.lax.broadcasted_iota(jnp.int32, (1, VB), 1)
    zm = jnp.where(col < VOCAB, z, -jnp.inf)
    m = jnp.max(zm)
    s = jnp.sum(jnp.exp(zm - m))
    out_ref[...] = z
    lane = jax.lax.broadcasted_iota(jnp.int32, (1, 8, 128), 2)
    stats_ref[...] = jnp.where(lane == 0, m, s)


def _normalize_kernel(stats_ref, z_ref, out_ref):
    m_all = stats_ref[:, 0, 0]
    s_all = stats_ref[:, 0, 1]
    m = jnp.max(m_all)
    norm = m + jnp.log(jnp.sum(s_all * jnp.exp(m_all - m)))
    out_ref[...] = z_ref[...] - norm


def kernel(inputs, emb_table, W1, b1, W2, b2):
    idx = inputs.astype(jnp.int32)

    logits, stats = pl.pallas_call(
        _logits_kernel,
        grid=(NB,),
        in_specs=[
            pl.BlockSpec(memory_space=pltpu.SMEM),
            pl.BlockSpec(memory_space=pltpu.MemorySpace.HBM),
            pl.BlockSpec(memory_space=pltpu.VMEM),
            pl.BlockSpec(memory_space=pltpu.VMEM),
            pl.BlockSpec((HID, VB), lambda i: (0, i)),
            pl.BlockSpec((1, VB), lambda i: (0, i)),
        ],
        out_specs=[
            pl.BlockSpec((1, VB), lambda i: (0, i)),
            pl.BlockSpec((1, 8, 128), lambda i: (i, 0, 0)),
        ],
        out_shape=[
            jax.ShapeDtypeStruct((1, VOCAB), jnp.float32),
            jax.ShapeDtypeStruct((NB, 8, 128), jnp.float32),
        ],
        scratch_shapes=[
            pltpu.VMEM((NCTX, D), jnp.float32),
            pltpu.SemaphoreType.DMA,
        ],
        compiler_params=pltpu.CompilerParams(
            dimension_semantics=("parallel",)),
    )(idx, emb_table, W1, b1.reshape(1, HID), W2, b2.reshape(1, VOCAB))

    return pl.pallas_call(
        _normalize_kernel,
        grid=(NB,),
        in_specs=[
            pl.BlockSpec(memory_space=pltpu.VMEM),
            pl.BlockSpec((1, VB), lambda i: (0, i)),
        ],
        out_specs=pl.BlockSpec((1, VB), lambda i: (0, i)),
        out_shape=jax.ShapeDtypeStruct((1, VOCAB), jnp.float32),
        compiler_params=pltpu.CompilerParams(
            dimension_semantics=("parallel",)),
    )(stats, logits)


# speedup vs baseline: 1.1605x; 1.1605x over previous
import jax
import jax.numpy as jnp
from jax.experimental import pallas as pl
from jax.experimental.pallas import tpu as pltpu

VOCAB = 100000
D = 64
NCTX = 20
HID = 128
VB = 8192
NB = VOCAB // VB
TAIL0 = NB * VB
TAIL = VOCAB - TAIL0
TB = 2048


def _head_kernel(idx_ref, emb_hbm, w1_ref, b1_ref, w2t_ref, b2t_ref,
                 h_ref, zt_ref, mt_ref, st_ref, rows_ref, row_sem):
    copies = [
        pltpu.make_async_copy(
            emb_hbm.at[pl.ds(idx_ref[k], 1), :],
            rows_ref.at[pl.ds(k, 1), :],
            row_sem,
        )
        for k in range(NCTX)
    ]
    for c in copies:
        c.start()
    for c in copies:
        c.wait()
    h = b1_ref[...]
    for k in range(NCTX):
        h = h + jnp.dot(rows_ref[pl.ds(k, 1), :],
                        w1_ref[pl.ds(k * D, D), :],
                        preferred_element_type=jnp.float32)
    h = jnp.maximum(h, 0.0)
    h_ref[...] = h

    zt = jnp.dot(h, w2t_ref[...],
                 preferred_element_type=jnp.float32) + b2t_ref[...]
    col = TAIL0 + jax.lax.broadcasted_iota(jnp.int32, (1, TB), 1)
    zt = jnp.where(col < VOCAB, zt, -jnp.inf)
    m = jnp.max(zt)
    s = jnp.sum(jnp.exp(zt - m))
    zt_ref[...] = zt
    mt_ref[...] = jnp.full((1, 128), m, jnp.float32)
    st_ref[...] = jnp.full((1, 128), s, jnp.float32)


def _stream_kernel(h_ref, w2_ref, b2_ref, z_ref, ms_ref, ss_ref):
    z = jnp.dot(h_ref[...], w2_ref[...],
                preferred_element_type=jnp.float32) + b2_ref[...]
    m = jnp.max(z)
    s = jnp.sum(jnp.exp(z - m))
    z_ref[...] = z
    ms_ref[...] = jnp.full((1, 8, 128), m, jnp.float32)
    ss_ref[...] = jnp.full((1, 8, 128), s, jnp.float32)


def _norm_kernel(ms_ref, ss_ref, mt_ref, st_ref, z_ref, zt_ref, out_ref):
    i = pl.program_id(0)
    ms = ms_ref[...]
    mt = jnp.max(mt_ref[...])
    M = jnp.maximum(jnp.max(ms), mt)
    S = (jnp.sum(ss_ref[...] * jnp.exp(ms - M)) / 1024.0
         + (jnp.sum(st_ref[...]) / 128.0) * jnp.exp(mt - M))
    norm = M + jnp.log(S)

    @pl.when(i < NB)
    def _():
        out_ref[...] = z_ref[...] - norm

    @pl.when(i == NB)
    def _():
        out_ref[:, :TB] = zt_ref[...] - norm


def kernel(inputs, emb_table, W1, b1, W2, b2):
    idx = inputs.astype(jnp.int32)
    b2r = b2.reshape(1, VOCAB)

    h, zt, mt, st = pl.pallas_call(
        _head_kernel,
        grid=(1,),
        in_specs=[
            pl.BlockSpec(memory_space=pltpu.SMEM),
            pl.BlockSpec(memory_space=pltpu.MemorySpace.HBM),
            pl.BlockSpec(memory_space=pltpu.VMEM),
            pl.BlockSpec(memory_space=pltpu.VMEM),
            pl.BlockSpec((HID, TB), lambda g: (0, TAIL0 // TB)),
            pl.BlockSpec((1, TB), lambda g: (0, TAIL0 // TB)),
        ],
        out_specs=[
            pl.BlockSpec(memory_space=pltpu.VMEM),
            pl.BlockSpec(memory_space=pltpu.VMEM),
            pl.BlockSpec(memory_space=pltpu.VMEM),
            pl.BlockSpec(memory_space=pltpu.VMEM),
        ],
        out_shape=[
            jax.ShapeDtypeStruct((1, HID), jnp.float32),
            jax.ShapeDtypeStruct((1, TB), jnp.float32),
            jax.ShapeDtypeStruct((1, 128), jnp.float32),
            jax.ShapeDtypeStruct((1, 128), jnp.float32),
        ],
        scratch_shapes=[pltpu.VMEM((NCTX, D), jnp.float32),
                        pltpu.SemaphoreType.DMA],
    )(idx, emb_table, W1, b1.reshape(1, HID), W2, b2r)

    z, ms, ss = pl.pallas_call(
        _stream_kernel,
        grid=(NB,),
        in_specs=[
            pl.BlockSpec((1, HID), lambda i: (0, 0)),
            pl.BlockSpec((HID, VB), lambda i: (0, i)),
            pl.BlockSpec((1, VB), lambda i: (0, i)),
        ],
        out_specs=[
            pl.BlockSpec((1, VB), lambda i: (0, i)),
            pl.BlockSpec((1, 8, 128), lambda i: (i, 0, 0)),
            pl.BlockSpec((1, 8, 128), lambda i: (i, 0, 0)),
        ],
        out_shape=[
            jax.ShapeDtypeStruct((1, TAIL0), jnp.float32),
            jax.ShapeDtypeStruct((NB, 8, 128), jnp.float32),
            jax.ShapeDtypeStruct((NB, 8, 128), jnp.float32),
        ],
        compiler_params=pltpu.CompilerParams(
            dimension_semantics=("parallel",)),
    )(h, W2, b2r)

    return pl.pallas_call(
        _norm_kernel,
        grid=(NB + 1,),
        in_specs=[
            pl.BlockSpec(memory_space=pltpu.VMEM),
            pl.BlockSpec(memory_space=pltpu.VMEM),
            pl.BlockSpec(memory_space=pltpu.VMEM),
            pl.BlockSpec(memory_space=pltpu.VMEM),
            pl.BlockSpec((1, VB), lambda i: (0, jnp.minimum(i, NB - 1))),
            pl.BlockSpec(memory_space=pltpu.VMEM),
        ],
        out_specs=pl.BlockSpec((1, VB), lambda i: (0, i)),
        out_shape=jax.ShapeDtypeStruct((1, VOCAB), jnp.float32),
        compiler_params=pltpu.CompilerParams(
            dimension_semantics=("parallel",)),
    )(ms, ss, mt, st, z, zt)


# speedup vs baseline: 1.1608x; 1.0003x over previous
import jax
import jax.numpy as jnp
from jax.experimental import pallas as pl
from jax.experimental.pallas import tpu as pltpu

VOCAB = 100000
D = 64
NCTX = 20
HID = 128
VB = 8192
NB = VOCAB // VB
NC = 2
NPC = NB // NC
HW = NPC * VB
TAIL0 = NB * VB
TAIL = VOCAB - TAIL0
TB = 2048


def _head_kernel(idx_ref, emb_hbm, w1_ref, b1_ref, w2t_ref, b2t_ref,
                 h_ref, zt_ref, mt_ref, st_ref, rows_ref, row_sem):
    copies = [
        pltpu.make_async_copy(
            emb_hbm.at[pl.ds(idx_ref[k], 1), :],
            rows_ref.at[pl.ds(k, 1), :],
            row_sem,
        )
        for k in range(NCTX)
    ]
    for c in copies:
        c.start()
    for c in copies:
        c.wait()
    h = b1_ref[...]
    for k in range(NCTX):
        h = h + jnp.dot(rows_ref[pl.ds(k, 1), :],
                        w1_ref[pl.ds(k * D, D), :],
                        preferred_element_type=jnp.float32)
    h = jnp.maximum(h, 0.0)
    h_ref[...] = h

    zt = jnp.dot(h, w2t_ref[...],
                 preferred_element_type=jnp.float32) + b2t_ref[...]
    col = TAIL0 + jax.lax.broadcasted_iota(jnp.int32, (1, TB), 1)
    zt = jnp.where(col < VOCAB, zt, -jnp.inf)
    m = jnp.max(zt)
    s = jnp.sum(jnp.exp(zt - m))
    zt_ref[...] = zt
    mt_ref[...] = jnp.full((1, 128), m, jnp.float32)
    st_ref[...] = jnp.full((1, 128), s, jnp.float32)


def _stream_kernel(h_ref, w2_hbm, b2_ref, z_ref, ms_ref, ss_ref,
                   wbuf_ref, sems):
    i = pl.program_id(0)
    for c in range(NC):
        @pl.when(i == c)
        def _(c=c):
            copies = [
                pltpu.make_async_copy(
                    w2_hbm.at[:, pl.ds((c * NPC + j) * VB, VB)],
                    wbuf_ref.at[j],
                    sems.at[j],
                )
                for j in range(NPC)
            ]
            for cp in copies:
                cp.start()
            h = h_ref[...]
            m = -jnp.inf
            s = 0.0
            for j in range(NPC):
                copies[j].wait()
                z = jnp.dot(h, wbuf_ref[j],
                            preferred_element_type=jnp.float32)
                z = z + b2_ref[:, j * VB:(j + 1) * VB]
                mn = jnp.maximum(m, jnp.max(z))
                s = s * jnp.exp(m - mn) + jnp.sum(jnp.exp(z - mn))
                m = mn
                z_ref[:, j * VB:(j + 1) * VB] = z
            ms_ref[...] = jnp.full((1, 8, 128), m, jnp.float32)
            ss_ref[...] = jnp.full((1, 8, 128), s, jnp.float32)


def _norm_kernel(ms_ref, ss_ref, mt_ref, st_ref, z_ref, zt_ref, out_ref):
    i = pl.program_id(0)
    ms = ms_ref[...]
    mt = jnp.max(mt_ref[...])
    M = jnp.maximum(jnp.max(ms), mt)
    S = (jnp.sum(ss_ref[...] * jnp.exp(ms - M)) / 1024.0
         + (jnp.sum(st_ref[...]) / 128.0) * jnp.exp(mt - M))
    norm = M + jnp.log(S)

    @pl.when(i < NB)
    def _():
        out_ref[...] = z_ref[...] - norm

    @pl.when(i == NB)
    def _():
        out_ref[:, :TB] = zt_ref[...] - norm


def kernel(inputs, emb_table, W1, b1, W2, b2):
    idx = inputs.astype(jnp.int32)
    b2r = b2.reshape(1, VOCAB)

    h, zt, mt, st = pl.pallas_call(
        _head_kernel,
        grid=(1,),
        in_specs=[
            pl.BlockSpec(memory_space=pltpu.SMEM),
            pl.BlockSpec(memory_space=pltpu.MemorySpace.HBM),
            pl.BlockSpec(memory_space=pltpu.VMEM),
            pl.BlockSpec(memory_space=pltpu.VMEM),
            pl.BlockSpec((HID, TB), lambda g: (0, TAIL0 // TB)),
            pl.BlockSpec((1, TB), lambda g: (0, TAIL0 // TB)),
        ],
        out_specs=[
            pl.BlockSpec(memory_space=pltpu.VMEM),
            pl.BlockSpec(memory_space=pltpu.VMEM),
            pl.BlockSpec(memory_space=pltpu.VMEM),
            pl.BlockSpec(memory_space=pltpu.VMEM),
        ],
        out_shape=[
            jax.ShapeDtypeStruct((1, HID), jnp.float32),
            jax.ShapeDtypeStruct((1, TB), jnp.float32),
            jax.ShapeDtypeStruct((1, 128), jnp.float32),
            jax.ShapeDtypeStruct((1, 128), jnp.float32),
        ],
        scratch_shapes=[pltpu.VMEM((NCTX, D), jnp.float32),
                        pltpu.SemaphoreType.DMA],
    )(idx, emb_table, W1, b1.reshape(1, HID), W2, b2r)

    z, ms, ss = pl.pallas_call(
        _stream_kernel,
        grid=(NC,),
        in_specs=[
            pl.BlockSpec((1, HID), lambda i: (0, 0)),
            pl.BlockSpec(memory_space=pltpu.MemorySpace.HBM),
            pl.BlockSpec((1, HW), lambda i: (0, i)),
        ],
        out_specs=[
            pl.BlockSpec((1, HW), lambda i: (0, i)),
            pl.BlockSpec((1, 8, 128), lambda i: (i, 0, 0)),
            pl.BlockSpec((1, 8, 128), lambda i: (i, 0, 0)),
        ],
        out_shape=[
            jax.ShapeDtypeStruct((1, TAIL0), jnp.float32),
            jax.ShapeDtypeStruct((NC, 8, 128), jnp.float32),
            jax.ShapeDtypeStruct((NC, 8, 128), jnp.float32),
        ],
        scratch_shapes=[pltpu.VMEM((NPC, HID, VB), jnp.float32),
                        pltpu.SemaphoreType.DMA((NPC,))],
        compiler_params=pltpu.CompilerParams(
            dimension_semantics=("parallel",)),
    )(h, W2, b2r)

    return pl.pallas_call(
        _norm_kernel,
        grid=(NB + 1,),
        in_specs=[
            pl.BlockSpec(memory_space=pltpu.VMEM),
            pl.BlockSpec(memory_space=pltpu.VMEM),
            pl.BlockSpec(memory_space=pltpu.VMEM),
            pl.BlockSpec(memory_space=pltpu.VMEM),
            pl.BlockSpec((1, VB), lambda i: (0, jnp.minimum(i, NB - 1))),
            pl.BlockSpec(memory_space=pltpu.VMEM),
        ],
        out_specs=pl.BlockSpec((1, VB), lambda i: (0, i)),
        out_shape=jax.ShapeDtypeStruct((1, VOCAB), jnp.float32),
        compiler_params=pltpu.CompilerParams(
            dimension_semantics=("parallel",)),
    )(ms, ss, mt, st, z, zt)


# speedup vs baseline: 1.2277x; 1.0576x over previous
import jax
import jax.numpy as jnp
from jax.experimental import pallas as pl
from jax.experimental.pallas import tpu as pltpu

VOCAB = 100000
D = 64
NCTX = 20
HID = 128
VB = 8192
NVB = VOCAB // VB
TAIL0 = NVB * VB
TAIL = VOCAB - TAIL0
TB = 2048


def _cbow_kernel(idx_ref, emb_hbm, w1_ref, b1_ref, w2_hbm, w2tail_ref,
                 b2_ref, out_ref, rows_ref, wbuf_ref, row_sem, w2_sem):
    row_copies = [
        pltpu.make_async_copy(
            emb_hbm.at[pl.ds(idx_ref[k], 1), :],
            rows_ref.at[pl.ds(k, 1), :],
            row_sem,
        )
        for k in range(NCTX)
    ]
    for c in row_copies:
        c.start()

    w2_copies = [
        pltpu.make_async_copy(
            w2_hbm.at[:, pl.ds(i * VB, VB)],
            wbuf_ref.at[i],
            w2_sem.at[i],
        )
        for i in range(NVB)
    ]
    for c in w2_copies:
        c.start()

    for c in row_copies:
        c.wait()
    h = b1_ref[...]
    for k in range(NCTX):
        h = h + jnp.dot(rows_ref[pl.ds(k, 1), :],
                        w1_ref[pl.ds(k * D, D), :],
                        preferred_element_type=jnp.float32)
    h = jnp.maximum(h, 0.0)

    zt = jnp.dot(h, w2tail_ref[...], preferred_element_type=jnp.float32)
    zt = zt[:, :TAIL] + b2_ref[:, TAIL0:]
    m = jnp.max(zt)
    s = jnp.sum(jnp.exp(zt - m))
    out_ref[:, TAIL0:] = zt

    for i in range(NVB):
        w2_copies[i].wait()
        z = jnp.dot(h, wbuf_ref[i], preferred_element_type=jnp.float32)
        z = z + b2_ref[:, i * VB:(i + 1) * VB]
        m_new = jnp.maximum(m, jnp.max(z))
        s = s * jnp.exp(m - m_new) + jnp.sum(jnp.exp(z - m_new))
        m = m_new
        out_ref[:, i * VB:(i + 1) * VB] = z

    out_ref[...] = out_ref[...] - (m + jnp.log(s))


def kernel(inputs, emb_table, W1, b1, W2, b2):
    idx = inputs.astype(jnp.int32)

    return pl.pallas_call(
        _cbow_kernel,
        grid=(1,),
        in_specs=[
            pl.BlockSpec(memory_space=pltpu.SMEM),
            pl.BlockSpec(memory_space=pltpu.MemorySpace.HBM),
            pl.BlockSpec(memory_space=pltpu.VMEM),
            pl.BlockSpec(memory_space=pltpu.VMEM),
            pl.BlockSpec(memory_space=pltpu.MemorySpace.HBM),
            pl.BlockSpec((HID, TB), lambda g: (0, TAIL0 // TB)),
            pl.BlockSpec(memory_space=pltpu.VMEM),
        ],
        out_specs=pl.BlockSpec(memory_space=pltpu.VMEM),
        out_shape=jax.ShapeDtypeStruct((1, VOCAB), jnp.float32),
        scratch_shapes=[
            pltpu.VMEM((NCTX, D), jnp.float32),
            pltpu.VMEM((NVB, HID, VB), jnp.float32),
            pltpu.SemaphoreType.DMA,
            pltpu.SemaphoreType.DMA((NVB,)),
        ],
    )(idx, emb_table, W1, b1.reshape(1, HID), W2, W2, b2.reshape(1, VOCAB))
